# Initial kernel scaffold; baseline (speedup 1.0000x reference)
import jax
import jax.numpy as jnp
from jax.experimental import pallas as pl


def kernel(x, mask, Wq, bq, Wk, bk, Wv, bv, Wo, bo, ln1_a, ln1_b, ln2_a, ln2_b, W1, b1, W2, b2):
    raise NotImplementedError("write your pallas kernel here")


# speedup vs baseline: 1.2881x; 1.2881x over previous
import functools
import math

import jax
import jax.numpy as jnp
from jax import lax
from jax.experimental import pallas as pl

N = 90
D_MODEL = 90
H = 6
D_K = 15
D_FF = 180
SEG = 128
PAD = SEG - N
WIDE = H * SEG

BB = 8


def _layer_norm(x, a, b, eps=1e-6):
    mean = jnp.mean(x, axis=-1, keepdims=True)
    var = jnp.sum((x - mean) ** 2, axis=-1, keepdims=True) / (x.shape[-1] - 1)
    std = jnp.sqrt(var)
    return a * (x - mean) / (std + eps) + b


def _gelu(x):
    c = math.sqrt(2.0 / math.pi)
    return 0.5 * x * (1.0 + jnp.tanh(c * (x + 0.044715 * x * x * x)))


def _block_kernel(x_ref, mask_ref, wq_ref, bq_ref, wk_ref, bk_ref, wv_ref,
                  bv_ref, wo_ref, bo_ref, ln1a_ref, ln1b_ref, ln2a_ref,
                  ln2b_ref, w1_ref, b1_ref, w2_ref, b2_ref,
                  out_ref, p_ref):
    f32 = jnp.float32
    x2 = x_ref[...].reshape(BB * N, D_MODEL)

    q = jnp.dot(x2, wq_ref[...], preferred_element_type=f32) + bq_ref[...]
    k = jnp.dot(x2, wk_ref[...], preferred_element_type=f32) + bk_ref[...]
    v = jnp.dot(x2, wv_ref[...], preferred_element_type=f32) + bv_ref[...]

    rows = lax.broadcasted_iota(jnp.int32, (WIDE, D_MODEL), 0)
    cols = lax.broadcasted_iota(jnp.int32, (WIDE, D_MODEL), 1)
    m6 = ((rows // SEG) == (cols // D_K)).astype(f32)

    zpad = jnp.zeros((PAD, D_MODEL), f32)
    inv_sqrt_dk = f32(1.0 / math.sqrt(D_K))
    neg_inf = jnp.full((N, PAD), -jnp.inf, f32)

    ctxs = []
    for g in range(BB):
        qg = q[g * N:(g + 1) * N, :]
        kg = k[g * N:(g + 1) * N, :]
        vg = v[g * N:(g + 1) * N, :]

        kpad = jnp.concatenate([kg, zpad], axis=0)
        kbig = jnp.concatenate([kpad] * H, axis=0) * m6
        s = lax.dot_general(qg, kbig, (((1,), (1,)), ((), ())),
                            preferred_element_type=f32) * inv_sqrt_dk

        mg = mask_ref[g]
        mpad = jnp.concatenate([mg, neg_inf], axis=1)
        s = s + jnp.concatenate([mpad] * H, axis=1)

        segs = []
        for h in range(H):
            seg = s[:, h * SEG:(h + 1) * SEG]
            mx = jnp.max(seg, axis=1, keepdims=True)
            e = jnp.exp(seg - mx)
            p = e / jnp.sum(e, axis=1, keepdims=True)
            p_ref[g, h] = p[:, :N]
            segs.append(p)
        pmat = jnp.concatenate(segs, axis=1)

        vpad = jnp.concatenate([vg, zpad], axis=0)
        vbig = jnp.concatenate([vpad] * H, axis=0) * m6
        ctxs.append(jnp.dot(pmat, vbig, preferred_element_type=f32))

    ctx = jnp.concatenate(ctxs, axis=0)
    attn = jnp.dot(ctx, wo_ref[...], preferred_element_type=f32) + bo_ref[...]
    x1 = attn + _layer_norm(attn, ln1a_ref[...], ln1b_ref[...])
    ffh = _gelu(jnp.dot(x1, w1_ref[...], preferred_element_type=f32)
                + b1_ref[...])
    ff = jnp.dot(ffh, w2_ref[...], preferred_element_type=f32) + b2_ref[...]
    out = ff + _layer_norm(ff, ln2a_ref[...], ln2b_ref[...])
    out_ref[...] = out.reshape(BB, N, D_MODEL)


@jax.jit
def kernel(x, mask, Wq, bq, Wk, bk, Wv, bv, Wo, bo, ln1_a, ln1_b, ln2_a,
           ln2_b, W1, b1, W2, b2):
    BT = x.shape[0]
    grid = (BT // BB,)

    def blk(i):
        return (i, 0, 0)

    def rep2(i):
        return (0, 0)

    in_specs = [
        pl.BlockSpec((BB, N, D_MODEL), blk),
        pl.BlockSpec((BB, N, N), blk),
        pl.BlockSpec((D_MODEL, D_MODEL), rep2),
        pl.BlockSpec((1, D_MODEL), rep2),
        pl.BlockSpec((D_MODEL, D_MODEL), rep2),
        pl.BlockSpec((1, D_MODEL), rep2),
        pl.BlockSpec((D_MODEL, D_MODEL), rep2),
        pl.BlockSpec((1, D_MODEL), rep2),
        pl.BlockSpec((D_MODEL, D_MODEL), rep2),
        pl.BlockSpec((1, D_MODEL), rep2),
        pl.BlockSpec((1, D_MODEL), rep2),
        pl.BlockSpec((1, D_MODEL), rep2),
        pl.BlockSpec((1, D_MODEL), rep2),
        pl.BlockSpec((1, D_MODEL), rep2),
        pl.BlockSpec((D_MODEL, D_FF), rep2),
        pl.BlockSpec((1, D_FF), rep2),
        pl.BlockSpec((D_FF, D_MODEL), rep2),
        pl.BlockSpec((1, D_MODEL), rep2),
    ]
    out_specs = [
        pl.BlockSpec((BB, N, D_MODEL), blk),
        pl.BlockSpec((BB, H, N, N), lambda i: (i, 0, 0, 0)),
    ]
    out_shapes = [
        jax.ShapeDtypeStruct((BT, N, D_MODEL), jnp.float32),
        jax.ShapeDtypeStruct((BT, H, N, N), jnp.float32),
    ]

    out, p_attn = pl.pallas_call(
        _block_kernel,
        grid=grid,
        in_specs=in_specs,
        out_specs=out_specs,
        out_shape=out_shapes,
    )(x, mask,
      Wq, bq.reshape(1, D_MODEL), Wk, bk.reshape(1, D_MODEL),
      Wv, bv.reshape(1, D_MODEL), Wo, bo.reshape(1, D_MODEL),
      ln1_a.reshape(1, D_MODEL), ln1_b.reshape(1, D_MODEL),
      ln2_a.reshape(1, D_MODEL), ln2_b.reshape(1, D_MODEL),
      W1, b1.reshape(1, D_FF), W2, b2.reshape(1, D_MODEL))
    return (out, p_attn)
